# baseline (speedup 1.0000x reference)
import jax
import jax.numpy as jnp
from jax import lax
from jax.experimental import pallas as pl
from jax.experimental.pallas import tpu as pltpu
from jax.experimental.pallas import tpu_sc as plsc

K = 3
N_C = 4096
N_F = 16384
D_FEAT = 256
D_RES = 256
D_OUT = 512
BF = 512
_BIG = 3.4e38
GW = 128
NI = K * N_F


def _select_block(pos_ref, pcTm2_ref, x2_ref, iota_ref, meta_ref):
    posb = pos_ref[...]
    pcTm2 = pcTm2_ref[...]
    xy2 = jnp.dot(posb, pcTm2, precision=lax.Precision.DEFAULT)
    s = posb[:, 3:4] + x2_ref[...]
    dist = s + xy2

    iota_row = iota_ref[...]
    amins = []
    for _ in range(K):
        amin = jnp.argmin(dist, axis=1).astype(jnp.float32)[:, None]
        amins.append(amin)
        dist = jnp.where(iota_row == amin, _BIG, dist)

    xy2_e = (posb[:, 0:1] * pcTm2[0:1, :]
             + posb[:, 1:2] * pcTm2[1:2, :]
             + posb[:, 2:3] * pcTm2[2:3, :])
    dex = s + xy2_e
    ds = [jnp.sum(jnp.where(iota_row == a, dex, 0.0), axis=1, keepdims=True)
          for a in amins]
    ws = [1.0 / jnp.maximum(d, 1e-16) for d in ds]
    wsum = ws[0] + ws[1] + ws[2]
    meta_ref[...] = jnp.concatenate(
        amins + [w / wsum for w in ws] + [jnp.zeros((BF, 2), jnp.float32)],
        axis=1)


@jax.jit
def _select(pos_pad, pcTm2, x2, iota_f):
    n = pos_pad.shape[0]
    grid = (n // BF,)
    return pl.pallas_call(
        _select_block,
        grid=grid,
        in_specs=[
            pl.BlockSpec((BF, 8), lambda i: (i, 0)),
            pl.BlockSpec((8, N_C), lambda i: (0, 0)),
            pl.BlockSpec((1, N_C), lambda i: (0, 0)),
            pl.BlockSpec((1, N_C), lambda i: (0, 0)),
        ],
        out_specs=pl.BlockSpec((BF, 8), lambda i: (i, 0)),
        out_shape=jax.ShapeDtypeStruct((n, 8), jnp.float32),
        compiler_params=pltpu.CompilerParams(
            dimension_semantics=("parallel",),
        ),
    )(pos_pad, pcTm2, x2, iota_f)


@jax.jit
def _sc_gather(x, indices):
    ni = indices.shape[1]
    mesh = plsc.VectorSubcoreMesh(core_axis_name="c", subcore_axis_name="s")

    @pl.kernel(out_type=jax.ShapeDtypeStruct((ni, D_FEAT), x.dtype), mesh=mesh)
    def body(x_hbm, i_hbm, o_hbm):
        def inner(i_vmem, o_vmem):
            pltpu.sync_copy(x_hbm.at[i_vmem.at[0]], o_vmem)

        pltpu.emit_pipeline(
            inner,
            grid=(ni // GW,),
            in_specs=[pl.BlockSpec((1, GW), index_map=lambda i: (0, i))],
            out_specs=[pl.BlockSpec((GW, D_FEAT), index_map=lambda i: (i, 0))],
            core_axis_name=("c", "s"),
            dimension_semantics=(pltpu.PARALLEL,),
        )(i_hbm, o_hbm)

    return body(x, indices)


def _combine_block(meta_ref, g0_ref, g1_ref, g2_ref, res_ref,
                   WxT_ref, WrT_ref, b_ref, out_ref):
    m = meta_ref[...]
    x = (m[:, 3:4] * g0_ref[...] + m[:, 4:5] * g1_ref[...]
         + m[:, 5:6] * g2_ref[...])
    out = (jnp.dot(x, WxT_ref[...], precision=lax.Precision.DEFAULT,
                   preferred_element_type=jnp.float32)
           + jnp.dot(res_ref[...], WrT_ref[...], precision=lax.Precision.DEFAULT,
                     preferred_element_type=jnp.float32)
           + b_ref[...])
    out_ref[...] = out


@jax.jit
def _combine(meta, g, res, WxT, WrT, b2):
    n = meta.shape[0]
    grid = (n // BF,)
    nblk = n // BF
    return pl.pallas_call(
        _combine_block,
        grid=grid,
        in_specs=[
            pl.BlockSpec((BF, 8), lambda i: (i, 0)),
            pl.BlockSpec((BF, D_FEAT), lambda i: (i, 0)),
            pl.BlockSpec((BF, D_FEAT), lambda i: (i + nblk, 0)),
            pl.BlockSpec((BF, D_FEAT), lambda i: (i + 2 * nblk, 0)),
            pl.BlockSpec((BF, D_RES), lambda i: (i, 0)),
            pl.BlockSpec((D_FEAT, D_OUT), lambda i: (0, 0)),
            pl.BlockSpec((D_RES, D_OUT), lambda i: (0, 0)),
            pl.BlockSpec((1, D_OUT), lambda i: (0, 0)),
        ],
        out_specs=pl.BlockSpec((BF, D_OUT), lambda i: (i, 0)),
        out_shape=jax.ShapeDtypeStruct((n, D_OUT), jnp.float32),
        compiler_params=pltpu.CompilerParams(
            dimension_semantics=("parallel",),
        ),
    )(meta, g, g, g, res, WxT, WrT, b2)


def kernel(x_c, pos_c, batch_c, res, pos, batch, W, b):
    y2 = jnp.sum(pos * pos, axis=1, keepdims=True)
    x2 = jnp.sum(pos_c * pos_c, axis=1).reshape(1, N_C)
    pos_pad = jnp.concatenate(
        [pos, y2, jnp.zeros((N_F, 4), jnp.float32)], axis=1)
    pcTm2 = jnp.pad((-2.0 * pos_c), ((0, 0), (0, 5))).T
    WxT = W[:, :D_FEAT].T.astype(jnp.bfloat16)
    WrT = W[:, D_FEAT:].T.astype(jnp.bfloat16)
    b2 = b.reshape(1, D_OUT)
    iota_f = jnp.arange(N_C, dtype=jnp.float32).reshape(1, N_C)

    h = N_F // 2
    metas, gs, outs = [], [], []
    for lo in (0, h):
        m = _select(pos_pad[lo:lo + h], pcTm2, x2, iota_f)
        metas.append(m)
        idx = m[:, :K].T.reshape(1, K * h).astype(jnp.int32)
        gs.append(_sc_gather(x_c, idx))
    for m, g, lo in zip(metas, gs, (0, h)):
        outs.append(_combine(m, g, res[lo:lo + h], WxT, WrT, b2))
    out = jnp.concatenate(outs, axis=0)
    return (out, pos, batch)


# speedup vs baseline: 1.0211x; 1.0211x over previous
import jax
import jax.numpy as jnp
from jax import lax
from jax.experimental import pallas as pl
from jax.experimental.pallas import tpu as pltpu
from jax.experimental.pallas import tpu_sc as plsc

K = 3
N_C = 4096
N_F = 16384
D_FEAT = 256
D_RES = 256
D_OUT = 512
BF = 512
_BIG = 3.4e38
GW = 128
NI = K * N_F


def _select_block(pos_ref, pcTm2_ref, x2_ref, iota_ref, meta_ref):
    posb = pos_ref[...]
    pcTm2 = pcTm2_ref[...]
    xy2 = jnp.dot(posb, pcTm2, precision=lax.Precision.DEFAULT)
    s = posb[:, 3:4] + x2_ref[...]
    dist = s + xy2

    iota_row = iota_ref[...]
    amins = []
    for _ in range(K):
        amin = jnp.argmin(dist, axis=1).astype(jnp.float32)[:, None]
        amins.append(amin)
        dist = jnp.where(iota_row == amin, _BIG, dist)

    xy2_e = (posb[:, 0:1] * pcTm2[0:1, :]
             + posb[:, 1:2] * pcTm2[1:2, :]
             + posb[:, 2:3] * pcTm2[2:3, :])
    dex = s + xy2_e
    ds = [jnp.sum(jnp.where(iota_row == a, dex, 0.0), axis=1, keepdims=True)
          for a in amins]
    ws = [1.0 / jnp.maximum(d, 1e-16) for d in ds]
    wsum = ws[0] + ws[1] + ws[2]
    meta_ref[...] = jnp.concatenate(
        amins + [w / wsum for w in ws] + [jnp.zeros((BF, 2), jnp.float32)],
        axis=1)


@jax.jit
def _select(pos_pad, pcTm2, x2, iota_f):
    grid = (N_F // BF,)
    return pl.pallas_call(
        _select_block,
        grid=grid,
        in_specs=[
            pl.BlockSpec((BF, 8), lambda i: (i, 0)),
            pl.BlockSpec((8, N_C), lambda i: (0, 0)),
            pl.BlockSpec((1, N_C), lambda i: (0, 0)),
            pl.BlockSpec((1, N_C), lambda i: (0, 0)),
        ],
        out_specs=pl.BlockSpec((BF, 8), lambda i: (i, 0)),
        out_shape=jax.ShapeDtypeStruct((N_F, 8), jnp.float32),
        compiler_params=pltpu.CompilerParams(
            dimension_semantics=("parallel",),
        ),
    )(pos_pad, pcTm2, x2, iota_f)


@jax.jit
def _sc_gather(x, indices):
    mesh = plsc.VectorSubcoreMesh(core_axis_name="c", subcore_axis_name="s")

    @pl.kernel(out_type=jax.ShapeDtypeStruct((NI, D_FEAT), x.dtype), mesh=mesh)
    def body(x_hbm, i_hbm, o_hbm):
        def inner(i_vmem, o_vmem):
            pltpu.sync_copy(x_hbm.at[i_vmem.at[0]], o_vmem)

        pltpu.emit_pipeline(
            inner,
            grid=(NI // GW,),
            in_specs=[pl.BlockSpec((1, GW), index_map=lambda i: (0, i))],
            out_specs=[pl.BlockSpec((GW, D_FEAT), index_map=lambda i: (i, 0))],
            core_axis_name=("c", "s"),
            dimension_semantics=(pltpu.PARALLEL,),
        )(i_hbm, o_hbm)

    return body(x, indices)


def _combine_block(meta_ref, g0_ref, g1_ref, g2_ref, res_ref,
                   WxT_ref, WrT_ref, b_ref, out_ref):
    m = meta_ref[...]
    x = (m[:, 3:4] * g0_ref[...] + m[:, 4:5] * g1_ref[...]
         + m[:, 5:6] * g2_ref[...])
    out = (jnp.dot(x, WxT_ref[...], precision=lax.Precision.DEFAULT,
                   preferred_element_type=jnp.float32)
           + jnp.dot(res_ref[...], WrT_ref[...], precision=lax.Precision.DEFAULT,
                     preferred_element_type=jnp.float32)
           + b_ref[...])
    out_ref[...] = out


@jax.jit
def _combine(meta, g, res, WxT, WrT, b2):
    grid = (N_F // BF,)
    nblk = N_F // BF
    return pl.pallas_call(
        _combine_block,
        grid=grid,
        in_specs=[
            pl.BlockSpec((BF, 8), lambda i: (i, 0)),
            pl.BlockSpec((BF, D_FEAT), lambda i: (i, 0)),
            pl.BlockSpec((BF, D_FEAT), lambda i: (i + nblk, 0)),
            pl.BlockSpec((BF, D_FEAT), lambda i: (i + 2 * nblk, 0)),
            pl.BlockSpec((BF, D_RES), lambda i: (i, 0)),
            pl.BlockSpec((D_FEAT, D_OUT), lambda i: (0, 0)),
            pl.BlockSpec((D_RES, D_OUT), lambda i: (0, 0)),
            pl.BlockSpec((1, D_OUT), lambda i: (0, 0)),
        ],
        out_specs=pl.BlockSpec((BF, D_OUT), lambda i: (i, 0)),
        out_shape=jax.ShapeDtypeStruct((N_F, D_OUT), jnp.float32),
        compiler_params=pltpu.CompilerParams(
            dimension_semantics=("parallel",),
        ),
    )(meta, g, g, g, res, WxT, WrT, b2)


def kernel(x_c, pos_c, batch_c, res, pos, batch, W, b):
    y2 = jnp.sum(pos * pos, axis=1, keepdims=True)
    x2 = jnp.sum(pos_c * pos_c, axis=1).reshape(1, N_C)
    pos_pad = jnp.concatenate(
        [pos, y2, jnp.zeros((N_F, 4), jnp.float32)], axis=1)
    pcTm2 = jnp.pad((-2.0 * pos_c), ((0, 0), (0, 5))).T
    WxT = W[:, :D_FEAT].T.astype(jnp.bfloat16)
    WrT = W[:, D_FEAT:].T.astype(jnp.bfloat16)
    b2 = b.reshape(1, D_OUT)
    iota_f = jnp.arange(N_C, dtype=jnp.float32).reshape(1, N_C)

    meta = _select(pos_pad, pcTm2, x2, iota_f)
    idx_cat = meta[:, :K].T.reshape(1, NI).astype(jnp.int32)
    g = _sc_gather(x_c, idx_cat)
    out = _combine(meta, g, res, WxT, WrT, b2)
    return (out, pos, batch)
